# baseline (speedup 1.0000x reference)
import functools

import jax
import jax.numpy as jnp
from jax import lax
from jax.experimental import pallas as pl
from jax.experimental.pallas import tpu as pltpu
from jax.experimental.pallas import tpu_sc as plsc

N = 10000
E = 640000
VD = 512
H = 32

PAD = 10240
LANES = 128
NC, NS = 2, 16
NW = NC * NS
EPW = 20480
EPAD = NW * EPW
ROWS_PER_W = EPW // LANES
NROWS = EPAD // LANES


def _mlp_body(x_ref, w1t_ref, b1_ref, g_ref, be_ref, a_ref, w2_ref, b2_ref,
              wc_ref, bc_ref, wp_ref, p_ref):
    x = x_ref[...]
    h = x[:, 0:1] * w1t_ref[0:1, :] + x[:, 1:2] * w1t_ref[1:2, :] + b1_ref[...]
    mu = jnp.mean(h, axis=0, keepdims=True)
    var = jnp.mean((h - mu) * (h - mu), axis=0, keepdims=True)
    h = (h - mu) / jnp.sqrt(var + 1e-5) * g_ref[...] + be_ref[...]
    a = a_ref[0, 0]
    h = jnp.where(h > 0, h, a * h)
    dn = (((1,), (1,)), ((), ()))
    h = lax.dot_general(h, w2_ref[...], dn,
                        preferred_element_type=jnp.float32) + b2_ref[...]
    h = lax.dot_general(h, wc_ref[...], dn,
                        preferred_element_type=jnp.float32) + bc_ref[...]
    p_ref[...] = lax.dot_general(h, wp_ref[...], dn,
                                 preferred_element_type=jnp.float32)


def _mlp_p(x, W1, b1, gamma, beta, prelu_a, W2, b2, Wc, bc, Wp):
    return pl.pallas_call(
        _mlp_body,
        out_shape=jax.ShapeDtypeStruct((N, 1), jnp.float32),
    )(x, W1.T, b1.reshape(1, H), gamma.reshape(1, H), beta.reshape(1, H),
      prelu_a.reshape(1, 1), W2, b2.reshape(1, H), Wc, bc.reshape(1, H), Wp)


_NB = 1280


def _norm_body(v_ref, vn_ref):
    v = v_ref[...]
    nrm = jnp.sqrt(jnp.sum(v * v, axis=1, keepdims=True))
    vn_ref[...] = v / (nrm + 1e-8)


def _norm(visual_pad):
    return pl.pallas_call(
        _norm_body,
        grid=(PAD // _NB,),
        in_specs=[pl.BlockSpec((_NB, VD), lambda i: (i, 0))],
        out_specs=pl.BlockSpec((_NB, VD), lambda i: (i, 0)),
        out_shape=jax.ShapeDtypeStruct((PAD, VD), jnp.float32),
    )(visual_pad)


_BR = 1024
_BC = 512
_NI = PAD // _BR
_NJ = PAD // _BC
_PAIR_I = [i for i in range(_NI) for j in range(2 * i, _NJ)]
_PAIR_J = [j for i in range(_NI) for j in range(2 * i, _NJ)]
_NPAIR = len(_PAIR_I)
_SLOT_BASE = [0] * _NI
for _i in range(1, _NI):
    _SLOT_BASE[_i] = _SLOT_BASE[_i - 1] + _NJ - 2 * (_i - 1)
_SLOT_W = _BR * _BC


def _mm_body(i_ref, j_ref, vn1_ref, vn2_ref, o_ref):
    r = lax.dot_general(
        vn1_ref[...], vn2_ref[...], (((1,), (1,)), ((), ())),
        preferred_element_type=jnp.float32)
    for t in range(_BC // LANES):
        o_ref[0, t * _BR:(t + 1) * _BR, :] = r[:, t * LANES:(t + 1) * LANES]


def _gram_tri(vn):
    iarr = jnp.asarray(_PAIR_I, dtype=jnp.int32)
    jarr = jnp.asarray(_PAIR_J, dtype=jnp.int32)
    return pl.pallas_call(
        _mm_body,
        grid_spec=pltpu.PrefetchScalarGridSpec(
            num_scalar_prefetch=2,
            grid=(_NPAIR,),
            in_specs=[
                pl.BlockSpec((_BR, VD), lambda b, I, J: (I[b], 0)),
                pl.BlockSpec((_BC, VD), lambda b, I, J: (J[b], 0)),
            ],
            out_specs=pl.BlockSpec(
                (1, (_BC // LANES) * _BR, LANES), lambda b, I, J: (b, 0, 0)),
        ),
        out_shape=jax.ShapeDtypeStruct(
            (_NPAIR, (_BC // LANES) * _BR, LANES), jnp.float32),
    )(iarr, jarr, vn, vn)


_NCH = 4
_CH = EPW // _NCH
_ZW = PAD // NS


def _sc_body(fi_hbm, src_hbm, dst_hbm, gp_hbm, p_hbm, out_hbm,
             fi0, fi1, fi2, fi3, r0, r1, r2, r3, d0, d1, d2, d3, w0, w1,
             p0, p1, zero_v, p_shared, shared, s0, s1, t0, t1):
    c = lax.axis_index("c")
    s = lax.axis_index("s")
    wid = c * NS + s

    def zb(i, carry):
        zero_v[pl.ds(i * 16, 16)] = jnp.zeros((16,), jnp.float32)
        return carry
    lax.fori_loop(0, _ZW // 16, zb, 0)
    pltpu.sync_copy(zero_v, shared.at[pl.ds(s * _ZW, _ZW)])

    @pl.when(s == 0)
    def _():
        pltpu.sync_copy(p_hbm, p_shared)

    plsc.subcore_barrier()

    base = wid * _NCH
    for j, (fv, rv, dv) in enumerate(((fi0, r0, d0), (fi1, r1, d1),
                                      (fi2, r2, d2), (fi3, r3, d3))):
        pltpu.sync_copy(fi_hbm.at[base + j], fv)
        pltpu.sync_copy(src_hbm.at[base + j], rv)
        pltpu.sync_copy(dst_hbm.at[base + j], dv)

    def pmul(pbuf, wbuf):
        def mb(k, carry):
            wbuf[pl.ds(k * 16, 16)] = (
                wbuf[pl.ds(k * 16, 16)] * pbuf[pl.ds(k * 16, 16)])
            return carry
        lax.fori_loop(0, _CH // 16, mb, 0)

    c0 = pltpu.async_copy(gp_hbm.at[fi0], w0, s0)
    g0 = pltpu.async_copy(p_shared.at[r0], p0, t0)
    c1 = pltpu.async_copy(gp_hbm.at[fi1], w1, s1)
    g1 = pltpu.async_copy(p_shared.at[r1], p1, t1)
    c0.wait(); g0.wait()
    pmul(p0, w0)
    pltpu.sync_copy(w0, shared.at[d0], add=True)
    c2 = pltpu.async_copy(gp_hbm.at[fi2], w0, s0)
    g2 = pltpu.async_copy(p_shared.at[r2], p0, t0)
    c1.wait(); g1.wait()
    pmul(p1, w1)
    pltpu.sync_copy(w1, shared.at[d1], add=True)
    c3 = pltpu.async_copy(gp_hbm.at[fi3], w1, s1)
    g3 = pltpu.async_copy(p_shared.at[r3], p1, t1)
    c2.wait(); g2.wait()
    pmul(p0, w0)
    pltpu.sync_copy(w0, shared.at[d2], add=True)
    c3.wait(); g3.wait()
    pmul(p1, w1)
    pltpu.sync_copy(w1, shared.at[d3], add=True)

    plsc.subcore_barrier()

    @pl.when(s == 0)
    def _():
        pltpu.sync_copy(shared, out_hbm.at[c])


_sc_scatter = functools.partial(
    pl.kernel,
    out_type=jax.ShapeDtypeStruct((NC, PAD), jnp.float32),
    mesh=plsc.VectorSubcoreMesh(
        core_axis_name="c", subcore_axis_name="s", num_cores=NC,
        num_subcores=NS),
    scratch_types=(
        [pltpu.VMEM((_CH,), jnp.int32)] * 12
        + [pltpu.VMEM((_CH,), jnp.float32)] * 4
        + [
            pltpu.VMEM((_ZW,), jnp.float32),
            pltpu.VMEM_SHARED((PAD,), jnp.float32),
            pltpu.VMEM_SHARED((PAD,), jnp.float32),
            pltpu.SemaphoreType.DMA,
            pltpu.SemaphoreType.DMA,
            pltpu.SemaphoreType.DMA,
            pltpu.SemaphoreType.DMA,
        ]
    ),
)(_sc_body)


def kernel(x, edge_index, visual, W1, b1, gamma, beta, prelu_a, W2, b2, Wc,
           bc, Wp, bp):
    p = _mlp_p(x, W1, b1, gamma, beta, prelu_a, W2, b2, Wc, bc, Wp)
    p_pad = jnp.pad(p, ((0, PAD - N), (0, 0))).reshape(PAD)

    visual_pad = jnp.pad(visual, ((0, PAD - N), (0, 0)))
    vn = _norm(visual_pad)
    gp = _gram_tri(vn).reshape(_NPAIR * _SLOT_W)

    src = edge_index[0].astype(jnp.int32)
    dst = edge_index[1].astype(jnp.int32)
    a = jnp.minimum(src, dst)
    b = jnp.maximum(src, dst)
    bi = a // _BR
    bj = b // _BC
    slot = jnp.take(jnp.asarray(_SLOT_BASE, dtype=jnp.int32), bi) \
        + bj - 2 * bi
    fi = slot * _SLOT_W + ((b % _BC) // LANES) * (_BR * LANES) \
        + (a % _BR) * LANES + b % LANES
    fi = jnp.pad(fi, (0, EPAD - E),
                 constant_values=_NPAIR * _SLOT_W - 1).reshape(
                     NW * _NCH, _CH)
    srcm = jnp.pad(src, (0, EPAD - E)).reshape(NW * _NCH, _CH)
    dstm = jnp.pad(dst, (0, EPAD - E)).reshape(NW * _NCH, _CH)

    parts = _sc_scatter(fi, srcm, dstm, gp, p_pad)
    return parts[0, :N] + parts[1, :N] + bp[0]


# speedup vs baseline: 1.2135x; 1.2135x over previous
import functools

import jax
import jax.numpy as jnp
from jax import lax
from jax.experimental import pallas as pl
from jax.experimental.pallas import tpu as pltpu
from jax.experimental.pallas import tpu_sc as plsc

N = 10000
E = 640000
VD = 512
H = 32

PAD = 10240
LANES = 128
NC, NS = 2, 16
NW = NC * NS
EPW = 20480
EPAD = NW * EPW


def _mlp_body(x_ref, w1t_ref, b1_ref, g_ref, be_ref, a_ref, w2_ref, b2_ref,
              wc_ref, bc_ref, wp_ref, p_ref):
    x = x_ref[...]
    h = x[:, 0:1] * w1t_ref[0:1, :] + x[:, 1:2] * w1t_ref[1:2, :] + b1_ref[...]
    mu = jnp.mean(h, axis=0, keepdims=True)
    var = jnp.mean((h - mu) * (h - mu), axis=0, keepdims=True)
    h = (h - mu) / jnp.sqrt(var + 1e-5) * g_ref[...] + be_ref[...]
    a = a_ref[0, 0]
    h = jnp.where(h > 0, h, a * h)
    dn = (((1,), (1,)), ((), ()))
    h = lax.dot_general(h, w2_ref[...], dn,
                        preferred_element_type=jnp.float32) + b2_ref[...]
    h = lax.dot_general(h, wc_ref[...], dn,
                        preferred_element_type=jnp.float32) + bc_ref[...]
    p_ref[...] = lax.dot_general(h, wp_ref[...], dn,
                                 preferred_element_type=jnp.float32)


def _mlp_p(x, W1, b1, gamma, beta, prelu_a, W2, b2, Wc, bc, Wp):
    return pl.pallas_call(
        _mlp_body,
        out_shape=jax.ShapeDtypeStruct((N, 1), jnp.float32),
    )(x, W1.T, b1.reshape(1, H), gamma.reshape(1, H), beta.reshape(1, H),
      prelu_a.reshape(1, 1), W2, b2.reshape(1, H), Wc, bc.reshape(1, H), Wp)


_NB = 1280


def _norm_body(v_ref, p_ref, vn_ref, u_ref):
    v = v_ref[...]
    nrm = jnp.sqrt(jnp.sum(v * v, axis=1, keepdims=True))
    vn = v / (nrm + 1e-8)
    vn_ref[...] = vn
    u_ref[...] = vn * p_ref[...]


def _norm_u(visual_pad, p_pad):
    return pl.pallas_call(
        _norm_body,
        grid=(PAD // _NB,),
        in_specs=[
            pl.BlockSpec((_NB, VD), lambda i: (i, 0)),
            pl.BlockSpec((_NB, 1), lambda i: (i, 0)),
        ],
        out_specs=[
            pl.BlockSpec((_NB, VD), lambda i: (i, 0)),
            pl.BlockSpec((_NB, VD), lambda i: (i, 0)),
        ],
        out_shape=[
            jax.ShapeDtypeStruct((PAD, VD), jnp.float32),
            jax.ShapeDtypeStruct((PAD, VD), jnp.float32),
        ],
    )(visual_pad, p_pad)


_BM = 256
_NT = PAD // LANES


def _mm_body(u_ref, vn_ref, o_ref):
    r = lax.dot_general(
        u_ref[...], vn_ref[...], (((1,), (1,)), ((), ())),
        preferred_element_type=jnp.float32)
    for t in range(_NT):
        o_ref[t] = r[:, t * LANES:(t + 1) * LANES]


def _gram(u, vn):
    return pl.pallas_call(
        _mm_body,
        grid=(PAD // _BM,),
        in_specs=[
            pl.BlockSpec((_BM, VD), lambda i: (i, 0)),
            pl.BlockSpec((PAD, VD), lambda i: (0, 0)),
        ],
        out_specs=pl.BlockSpec((_NT, _BM, LANES), lambda i: (0, i, 0)),
        out_shape=jax.ShapeDtypeStruct((_NT, PAD, LANES), jnp.float32),
    )(u, vn)


_NCH = 4
_CH = EPW // _NCH
_ZW = PAD // NS


def _sc_body(fi_hbm, dst_hbm, gp_hbm, out_hbm, fi0, fi1, fi2, fi3,
             d0, d1, d2, d3, w0, w1, zero_v, shared, s0, s1):
    c = lax.axis_index("c")
    s = lax.axis_index("s")
    wid = c * NS + s

    def zb(i, carry):
        zero_v[pl.ds(i * 16, 16)] = jnp.zeros((16,), jnp.float32)
        return carry
    lax.fori_loop(0, _ZW // 16, zb, 0)
    pltpu.sync_copy(zero_v, shared.at[pl.ds(s * _ZW, _ZW)])
    plsc.subcore_barrier()

    base = wid * _NCH
    for j, (fv, dv) in enumerate(((fi0, d0), (fi1, d1), (fi2, d2),
                                  (fi3, d3))):
        pltpu.sync_copy(fi_hbm.at[base + j], fv)
        pltpu.sync_copy(dst_hbm.at[base + j], dv)

    c0 = pltpu.async_copy(gp_hbm.at[fi0], w0, s0)
    c1 = pltpu.async_copy(gp_hbm.at[fi1], w1, s1)
    c0.wait()
    pltpu.sync_copy(w0, shared.at[d0], add=True)
    c2 = pltpu.async_copy(gp_hbm.at[fi2], w0, s0)
    c1.wait()
    pltpu.sync_copy(w1, shared.at[d1], add=True)
    c3 = pltpu.async_copy(gp_hbm.at[fi3], w1, s1)
    c2.wait()
    pltpu.sync_copy(w0, shared.at[d2], add=True)
    c3.wait()
    pltpu.sync_copy(w1, shared.at[d3], add=True)

    plsc.subcore_barrier()

    @pl.when(s == 0)
    def _():
        pltpu.sync_copy(shared, out_hbm.at[c])


_sc_scatter = functools.partial(
    pl.kernel,
    out_type=jax.ShapeDtypeStruct((NC, PAD), jnp.float32),
    mesh=plsc.VectorSubcoreMesh(
        core_axis_name="c", subcore_axis_name="s", num_cores=NC,
        num_subcores=NS),
    scratch_types=(
        [pltpu.VMEM((_CH,), jnp.int32)] * 8
        + [pltpu.VMEM((_CH,), jnp.float32)] * 2
        + [
            pltpu.VMEM((_ZW,), jnp.float32),
            pltpu.VMEM_SHARED((PAD,), jnp.float32),
            pltpu.SemaphoreType.DMA,
            pltpu.SemaphoreType.DMA,
        ]
    ),
)(_sc_body)


def kernel(x, edge_index, visual, W1, b1, gamma, beta, prelu_a, W2, b2, Wc,
           bc, Wp, bp):
    p = _mlp_p(x, W1, b1, gamma, beta, prelu_a, W2, b2, Wc, bc, Wp)

    visual_pad = jnp.pad(visual, ((0, PAD - N), (0, 0)))
    p_pad = jnp.pad(p, ((0, PAD - N), (0, 0)))
    vn, u = _norm_u(visual_pad, p_pad)
    gp = _gram(u, vn).reshape(PAD * PAD)

    src = edge_index[0].astype(jnp.int32)
    dst = edge_index[1].astype(jnp.int32)
    fi = (dst // LANES) * (PAD * LANES) + src * LANES + dst % LANES
    fi = jnp.pad(fi, (0, EPAD - E),
                 constant_values=PAD * PAD - 1).reshape(NW * _NCH, _CH)
    dstm = jnp.pad(dst, (0, EPAD - E)).reshape(NW * _NCH, _CH)

    parts = _sc_scatter(fi, dstm, gp)
    return parts[0, :N] + parts[1, :N] + bp[0]
